# baseline (speedup 1.0000x reference)
import numpy as np
import jax
import jax.numpy as jnp
from jax.experimental import pallas as pl
from jax.experimental.pallas import tpu as pltpu

_NK1 = 0x03D7B32D
_NK2 = 0xADD083F4

_ROT = ((13, 15, 26, 6), (17, 29, 16, 24))


def _threefry_bits(idx):
    k1 = jnp.uint32(_NK1)
    k2 = jnp.uint32(_NK2)
    ks2 = jnp.uint32(_NK1 ^ _NK2 ^ 0x1BD11BDA)
    ks = (k1, k2, ks2)
    x0 = jnp.full(idx.shape, k1, dtype=jnp.uint32)
    x1 = idx + k2
    for i in range(5):
        for r in _ROT[i % 2]:
            x0 = x0 + x1
            x1 = (x1 << jnp.uint32(r)) | (x1 >> jnp.uint32(32 - r))
            x1 = x1 ^ x0
        x0 = x0 + ks[(i + 1) % 3]
        x1 = x1 + ks[(i + 2) % 3] + jnp.uint32(i + 1)
    return x0 ^ x1


def _noise_term(idx):
    bits = _threefry_bits(idx)
    fb = (bits >> jnp.uint32(9)) | jnp.uint32(0x3F800000)
    f = jax.lax.bitcast_convert_type(fb, jnp.float32) - jnp.float32(1.0)
    lo = jnp.float32(np.nextafter(np.float32(-1.0), np.float32(0.0)))
    hi = jnp.float32(1.0)
    u = jnp.maximum(lo, f * (hi - lo) + lo)
    w = -jnp.log(jnp.float32(1.0) - u * u)
    t = jnp.sqrt(w) - jnp.float32(3.0)
    p = jnp.float32(-0.000200214257)
    for c in (0.000100950558, 0.00134934322, -0.00367342844, 0.00573950773,
              -0.0076224613, 0.00943887047, 1.00167406, 2.83297682):
        p = jnp.float32(c) + p * t
    noise = jnp.float32(np.sqrt(2.0)) * (p * u)
    term = jnp.maximum(noise - jnp.float32(3.0), jnp.float32(0.0))
    return jnp.where(w >= jnp.float32(5.0), term, jnp.float32(0.0))


def _p1_reduce(a_ref, s_ref):
    part = jnp.sum(jnp.abs(a_ref[...]), axis=0, keepdims=True)

    @pl.when(pl.program_id(0) == 0)
    def _():
        s_ref[...] = part

    @pl.when(pl.program_id(0) != 0)
    def _():
        s_ref[...] = s_ref[...] + part


def _p2_sample(nheads, batch, reps):
    def body(s_ref, gum_ref, hmod_ref, mask_ref, ac_ref):
        x = s_ref[...]
        sh = nheads
        while sh < x.shape[-1]:
            x = x + jnp.roll(x, -sh, axis=1)
            sh *= 2
        activ = jnp.float32(0.03) * x
        ac_ref[...] = jnp.exp(jnp.float32(-5.0) * activ)
        ac = ac_ref[...]
        tot = jnp.sum(ac) / jnp.float32(reps)
        lane = jax.lax.broadcasted_iota(jnp.int32, x.shape, 1)
        base = jnp.log(ac)
        last = jnp.log((tot * jnp.float32(99.0) * jnp.float32(batch))
                       * jnp.ones_like(base))
        scores = jnp.where(lane == nheads, last, base)
        scores = jnp.where(lane <= nheads, scores, -jnp.inf)
        scores = scores + gum_ref[...]
        m = jnp.max(scores)
        r = jnp.min(jnp.where(scores == m, lane, jnp.int32(2**30)))
        mask_ref[...] = jnp.where(hmod_ref[...] == r, jnp.float32(3.0),
                                  jnp.float32(0.0))
    return body


def _p3_apply(block_rows, lanes):
    def body(a_ref, mask_ref, o_ref):
        i = pl.program_id(0)
        row = jax.lax.broadcasted_iota(jnp.int32, (block_rows, lanes), 0)
        col = jax.lax.broadcasted_iota(jnp.int32, (block_rows, lanes), 1)
        flat = (i * block_rows + row) * lanes + col
        term = _noise_term(flat.astype(jnp.uint32))
        o_ref[...] = a_ref[...] + term * mask_ref[...]
    return body


def kernel(a):
    batch, ntok, ntok2, nheads = a.shape
    total = batch * ntok * ntok2 * nheads
    lanes = nheads * 128
    rows = total // lanes
    reps = lanes // nheads
    a2 = a.reshape(rows, lanes)

    br1 = 1024
    s_cols = pl.pallas_call(
        _p1_reduce,
        grid=(rows // br1,),
        in_specs=[pl.BlockSpec((br1, lanes), lambda i: (i, 0))],
        out_specs=pl.BlockSpec((1, lanes), lambda i: (0, 0)),
        out_shape=jax.ShapeDtypeStruct((1, lanes), jnp.float32),
    )(a2)

    gum = jnp.zeros((1, lanes), jnp.float32)
    gum = gum.at[0, : nheads + 1].set(
        jax.random.gumbel(jax.random.key(42), (nheads + 1,), jnp.float32))
    hmod = jax.lax.broadcasted_iota(jnp.int32, (1, lanes), 1) % nheads

    mask = pl.pallas_call(
        _p2_sample(nheads, batch, reps),
        grid=(1,),
        in_specs=[pl.BlockSpec((1, lanes), lambda i: (0, 0))] * 3,
        out_specs=pl.BlockSpec((1, lanes), lambda i: (0, 0)),
        out_shape=jax.ShapeDtypeStruct((1, lanes), jnp.float32),
        scratch_shapes=[pltpu.VMEM((1, lanes), jnp.float32)],
    )(s_cols, gum, hmod)

    br3 = 256
    out2 = pl.pallas_call(
        _p3_apply(br3, lanes),
        grid=(rows // br3,),
        in_specs=[pl.BlockSpec((br3, lanes), lambda i: (i, 0)),
                  pl.BlockSpec((1, lanes), lambda i: (0, 0))],
        out_specs=pl.BlockSpec((br3, lanes), lambda i: (i, 0)),
        out_shape=jax.ShapeDtypeStruct((rows, lanes), jnp.float32),
    )(a2, mask)

    return out2.reshape(a.shape)


# speedup vs baseline: 19.9855x; 19.9855x over previous
import numpy as np
import jax
import jax.numpy as jnp
from jax.experimental import pallas as pl
from jax.experimental.pallas import tpu as pltpu

_NK1 = 0x03D7B32D
_NK2 = 0xADD083F4

_ROT = ((13, 15, 26, 6), (17, 29, 16, 24))


def _threefry_bits(idx):
    k1 = jnp.uint32(_NK1)
    k2 = jnp.uint32(_NK2)
    ks2 = jnp.uint32(_NK1 ^ _NK2 ^ 0x1BD11BDA)
    ks = (k1, k2, ks2)
    x0 = jnp.full(idx.shape, k1, dtype=jnp.uint32)
    x1 = idx + k2
    for i in range(5):
        for r in _ROT[i % 2]:
            x0 = x0 + x1
            x1 = (x1 << jnp.uint32(r)) | (x1 >> jnp.uint32(32 - r))
            x1 = x1 ^ x0
        x0 = x0 + ks[(i + 1) % 3]
        x1 = x1 + ks[(i + 2) % 3] + jnp.uint32(i + 1)
    return x0 ^ x1


def _noise_term(idx):
    bits = _threefry_bits(idx)
    fb = (bits >> jnp.uint32(9)) | jnp.uint32(0x3F800000)
    f = jax.lax.bitcast_convert_type(fb, jnp.float32) - jnp.float32(1.0)
    lo = jnp.float32(np.nextafter(np.float32(-1.0), np.float32(0.0)))
    hi = jnp.float32(1.0)
    u = jnp.maximum(lo, f * (hi - lo) + lo)
    w = -jnp.log(jnp.float32(1.0) - u * u)
    t = jnp.sqrt(w) - jnp.float32(3.0)
    p = jnp.float32(-0.000200214257)
    for c in (0.000100950558, 0.00134934322, -0.00367342844, 0.00573950773,
              -0.0076224613, 0.00943887047, 1.00167406, 2.83297682):
        p = jnp.float32(c) + p * t
    noise = jnp.float32(np.sqrt(2.0)) * (p * u)
    term = jnp.maximum(noise - jnp.float32(3.0), jnp.float32(0.0))
    return jnp.where(w >= jnp.float32(5.0), term, jnp.float32(0.0))


def _p1_reduce(a_ref, s_ref):
    part = jnp.sum(jnp.abs(a_ref[...]), axis=1, keepdims=True)

    @pl.when(pl.program_id(0) == 0)
    def _():
        s_ref[...] = part

    @pl.when(pl.program_id(0) != 0)
    def _():
        s_ref[...] = s_ref[...] + part


def _p2_sample(nheads, batch):
    def body(s_ref, gum_ref, r_ref, ac_ref):
        x = s_ref[...][:, 0, :]
        tr = jnp.transpose(x)
        s12 = jnp.sum(tr, axis=0, keepdims=True)
        s128 = jnp.pad(s12, ((0, 0), (0, 128 - nheads)))
        lane = jax.lax.broadcasted_iota(jnp.int32, s128.shape, 1)
        activ = jnp.float32(0.03) * s128
        ac_ref[...] = jnp.exp(jnp.float32(-5.0) * activ)
        ac = ac_ref[...]
        tot = jnp.sum(jnp.where(lane < nheads, ac, jnp.float32(0.0)))
        base = jnp.log(ac)
        last = jnp.log((tot * jnp.float32(99.0) * jnp.float32(batch))
                       * jnp.ones_like(base))
        scores = jnp.where(lane == nheads, last, base)
        scores = jnp.where(lane <= nheads, scores, -jnp.inf)
        scores = scores + gum_ref[...]
        m = jnp.max(scores)
        r = jnp.min(jnp.where(scores == m, lane, jnp.int32(2 ** 30)))
        r_ref[...] = jnp.full(r_ref.shape, r, jnp.int32)
    return body


def _p3_apply(block_rows, ntok, nheads):
    def body(r_ref, a_ref, o_ref):
        h = pl.program_id(0)
        i = pl.program_id(1)
        r = r_ref[0]

        @pl.when(h == r)
        def _():
            t1 = jax.lax.broadcasted_iota(jnp.int32, (1, block_rows, ntok), 1)
            t2 = jax.lax.broadcasted_iota(jnp.int32, (1, block_rows, ntok), 2)
            flat = ((i * block_rows + t1) * ntok + t2) * nheads + r
            term = _noise_term(flat.astype(jnp.uint32))
            o_ref[...] = a_ref[...] + jnp.float32(3.0) * term

        @pl.when(h != r)
        def _():
            o_ref[...] = a_ref[...]
    return body


def kernel(a):
    batch, ntok, ntok2, nheads = a.shape
    av = jnp.transpose(a, (0, 3, 1, 2)).reshape(nheads, ntok, ntok2)

    br1 = 64
    s_hc = pl.pallas_call(
        _p1_reduce,
        grid=(ntok // br1,),
        in_specs=[pl.BlockSpec((nheads, br1, ntok2), lambda i: (0, i, 0))],
        out_specs=pl.BlockSpec((nheads, 1, ntok2), lambda i: (0, 0, 0)),
        out_shape=jax.ShapeDtypeStruct((nheads, 1, ntok2), jnp.float32),
    )(av)

    gum = jnp.zeros((1, 128), jnp.float32)
    gum = gum.at[0, : nheads + 1].set(
        jax.random.gumbel(jax.random.key(42), (nheads + 1,), jnp.float32))

    r_plane = pl.pallas_call(
        _p2_sample(nheads, batch),
        grid=(1,),
        in_specs=[pl.BlockSpec((nheads, 1, ntok2), lambda i: (0, 0, 0)),
                  pl.BlockSpec((1, 128), lambda i: (0, 0))],
        out_specs=pl.BlockSpec((8, 128), lambda i: (0, 0)),
        out_shape=jax.ShapeDtypeStruct((8, 128), jnp.int32),
        scratch_shapes=[pltpu.VMEM((1, 128), jnp.float32)],
    )(s_hc, gum)
    r_arr = r_plane[0, :1]

    br3 = 256
    outp = pl.pallas_call(
        _p3_apply(br3, ntok2, nheads),
        grid_spec=pltpu.PrefetchScalarGridSpec(
            num_scalar_prefetch=1,
            grid=(nheads, ntok // br3),
            in_specs=[pl.BlockSpec((1, br3, ntok2), lambda h, i, r: (h, i, 0))],
            out_specs=pl.BlockSpec((1, br3, ntok2), lambda h, i, r: (h, i, 0)),
        ),
        out_shape=jax.ShapeDtypeStruct((nheads, ntok, ntok2), jnp.float32),
    )(r_arr, av)

    out = jnp.transpose(outp.reshape(batch, nheads, ntok, ntok2),
                        (0, 2, 3, 1))
    return out


# speedup vs baseline: 19.9938x; 1.0004x over previous
import numpy as np
import jax
import jax.numpy as jnp
from jax.experimental import pallas as pl
from jax.experimental.pallas import tpu as pltpu

_NK1 = 0x03D7B32D
_NK2 = 0xADD083F4

_ROT = ((13, 15, 26, 6), (17, 29, 16, 24))


def _threefry_bits(idx):
    k1 = jnp.uint32(_NK1)
    k2 = jnp.uint32(_NK2)
    ks2 = jnp.uint32(_NK1 ^ _NK2 ^ 0x1BD11BDA)
    ks = (k1, k2, ks2)
    x0 = jnp.full(idx.shape, k1, dtype=jnp.uint32)
    x1 = idx + k2
    for i in range(5):
        for r in _ROT[i % 2]:
            x0 = x0 + x1
            x1 = (x1 << jnp.uint32(r)) | (x1 >> jnp.uint32(32 - r))
            x1 = x1 ^ x0
        x0 = x0 + ks[(i + 1) % 3]
        x1 = x1 + ks[(i + 2) % 3] + jnp.uint32(i + 1)
    return x0 ^ x1


def _noise_term(idx):
    bits = _threefry_bits(idx)
    fb = (bits >> jnp.uint32(9)) | jnp.uint32(0x3F800000)
    f = jax.lax.bitcast_convert_type(fb, jnp.float32) - jnp.float32(1.0)
    lo = jnp.float32(np.nextafter(np.float32(-1.0), np.float32(0.0)))
    hi = jnp.float32(1.0)
    u = jnp.maximum(lo, f * (hi - lo) + lo)
    w = -jnp.log(jnp.float32(1.0) - u * u)
    t = jnp.sqrt(w) - jnp.float32(3.0)
    p = jnp.float32(-0.000200214257)
    for c in (0.000100950558, 0.00134934322, -0.00367342844, 0.00573950773,
              -0.0076224613, 0.00943887047, 1.00167406, 2.83297682):
        p = jnp.float32(c) + p * t
    noise = jnp.float32(np.sqrt(2.0)) * (p * u)
    term = jnp.maximum(noise - jnp.float32(3.0), jnp.float32(0.0))
    return jnp.where(w >= jnp.float32(5.0), term, jnp.float32(0.0))


def _p1_reduce(a_ref, s_ref):
    part = jnp.sum(jnp.abs(a_ref[...]), axis=1, keepdims=True)

    @pl.when(pl.program_id(0) == 0)
    def _():
        s_ref[...] = part

    @pl.when(pl.program_id(0) != 0)
    def _():
        s_ref[...] = s_ref[...] + part


def _p2_sample(nheads, batch):
    def body(s_ref, gum_ref, r_ref, ac_ref):
        x = s_ref[...][:, 0, :]
        tr = jnp.transpose(x)
        s12 = jnp.sum(tr, axis=0, keepdims=True)
        s128 = jnp.pad(s12, ((0, 0), (0, 128 - nheads)))
        lane = jax.lax.broadcasted_iota(jnp.int32, s128.shape, 1)
        activ = jnp.float32(0.03) * s128
        ac_ref[...] = jnp.exp(jnp.float32(-5.0) * activ)
        ac = ac_ref[...]
        tot = jnp.sum(jnp.where(lane < nheads, ac, jnp.float32(0.0)))
        base = jnp.log(ac)
        last = jnp.log((tot * jnp.float32(99.0) * jnp.float32(batch))
                       * jnp.ones_like(base))
        scores = jnp.where(lane == nheads, last, base)
        scores = jnp.where(lane <= nheads, scores, -jnp.inf)
        scores = scores + gum_ref[...]
        m = jnp.max(scores)
        r = jnp.min(jnp.where(scores == m, lane, jnp.int32(2 ** 30)))
        r_ref[...] = jnp.full(r_ref.shape, r, jnp.int32)
    return body


def _p3_update(block_rows, ntok, nheads):
    def body(r_ref, x_ref, o_ref):
        i = pl.program_id(0)
        r = r_ref[0]
        rr = jnp.minimum(r, nheads - 1)
        t1 = jax.lax.broadcasted_iota(jnp.int32, (1, block_rows, ntok), 1)
        t2 = jax.lax.broadcasted_iota(jnp.int32, (1, block_rows, ntok), 2)
        flat = ((i * block_rows + t1) * ntok + t2) * nheads + rr
        term = _noise_term(flat.astype(jnp.uint32))
        scale = jnp.where(r < nheads, jnp.float32(3.0), jnp.float32(0.0))
        o_ref[...] = x_ref[...] + scale * term
    return body


def kernel(a):
    batch, ntok, ntok2, nheads = a.shape
    av = jnp.transpose(a, (0, 3, 1, 2)).reshape(nheads, ntok, ntok2)

    br1 = 64
    s_hc = pl.pallas_call(
        _p1_reduce,
        grid=(ntok // br1,),
        in_specs=[pl.BlockSpec((nheads, br1, ntok2), lambda i: (0, i, 0))],
        out_specs=pl.BlockSpec((nheads, 1, ntok2), lambda i: (0, 0, 0)),
        out_shape=jax.ShapeDtypeStruct((nheads, 1, ntok2), jnp.float32),
    )(av)

    gum = jnp.zeros((1, 128), jnp.float32)
    gum = gum.at[0, : nheads + 1].set(
        jax.random.gumbel(jax.random.key(42), (nheads + 1,), jnp.float32))

    r_plane = pl.pallas_call(
        _p2_sample(nheads, batch),
        grid=(1,),
        in_specs=[pl.BlockSpec((nheads, 1, ntok2), lambda i: (0, 0, 0)),
                  pl.BlockSpec((1, 128), lambda i: (0, 0))],
        out_specs=pl.BlockSpec((8, 128), lambda i: (0, 0)),
        out_shape=jax.ShapeDtypeStruct((8, 128), jnp.int32),
        scratch_shapes=[pltpu.VMEM((1, 128), jnp.float32)],
    )(s_hc, gum)
    r_arr = r_plane[0, :1]

    br3 = 256
    hmax = nheads - 1

    def _rblock(i, r):
        return (jnp.minimum(r[0], hmax), i, 0)

    outp = pl.pallas_call(
        _p3_update(br3, ntok2, nheads),
        grid_spec=pltpu.PrefetchScalarGridSpec(
            num_scalar_prefetch=1,
            grid=(ntok // br3,),
            in_specs=[pl.BlockSpec((1, br3, ntok2), _rblock)],
            out_specs=pl.BlockSpec((1, br3, ntok2), _rblock),
        ),
        out_shape=jax.ShapeDtypeStruct((nheads, ntok, ntok2), jnp.float32),
        input_output_aliases={1: 0},
    )(r_arr, av)

    out = jnp.transpose(outp.reshape(batch, nheads, ntok, ntok2),
                        (0, 2, 3, 1))
    return out


# speedup vs baseline: 24.9924x; 1.2500x over previous
import numpy as np
import jax
import jax.numpy as jnp
from jax.experimental import pallas as pl
from jax.experimental.pallas import tpu as pltpu

_NK1 = 0x03D7B32D
_NK2 = 0xADD083F4

_ROT = ((13, 15, 26, 6), (17, 29, 16, 24))


def _threefry_bits(idx):
    k1 = jnp.uint32(_NK1)
    k2 = jnp.uint32(_NK2)
    ks2 = jnp.uint32(_NK1 ^ _NK2 ^ 0x1BD11BDA)
    ks = (k1, k2, ks2)
    x0 = jnp.full(idx.shape, k1, dtype=jnp.uint32)
    x1 = idx + k2
    for i in range(5):
        for r in _ROT[i % 2]:
            x0 = x0 + x1
            x1 = (x1 << jnp.uint32(r)) | (x1 >> jnp.uint32(32 - r))
            x1 = x1 ^ x0
        x0 = x0 + ks[(i + 1) % 3]
        x1 = x1 + ks[(i + 2) % 3] + jnp.uint32(i + 1)
    return x0 ^ x1


def _noise_term(idx):
    bits = _threefry_bits(idx)
    fb = (bits >> jnp.uint32(9)) | jnp.uint32(0x3F800000)
    f = jax.lax.bitcast_convert_type(fb, jnp.float32) - jnp.float32(1.0)
    lo = jnp.float32(np.nextafter(np.float32(-1.0), np.float32(0.0)))
    hi = jnp.float32(1.0)
    u = jnp.maximum(lo, f * (hi - lo) + lo)
    w = -jnp.log(jnp.float32(1.0) - u * u)
    t = jnp.sqrt(w) - jnp.float32(3.0)
    p = jnp.float32(-0.000200214257)
    for c in (0.000100950558, 0.00134934322, -0.00367342844, 0.00573950773,
              -0.0076224613, 0.00943887047, 1.00167406, 2.83297682):
        p = jnp.float32(c) + p * t
    noise = jnp.float32(np.sqrt(2.0)) * (p * u)
    term = jnp.maximum(noise - jnp.float32(3.0), jnp.float32(0.0))
    return jnp.where(w >= jnp.float32(5.0), term, jnp.float32(0.0))


def _p1_reduce_copy(a_ref, s_ref, o_ref):
    x = a_ref[...]
    o_ref[...] = x
    part = jnp.sum(jnp.abs(x), axis=1, keepdims=True)

    @pl.when(pl.program_id(0) == 0)
    def _():
        s_ref[...] = part

    @pl.when(pl.program_id(0) != 0)
    def _():
        s_ref[...] = s_ref[...] + part


def _p2_sample(nheads, batch):
    def body(s_ref, gum_ref, r_ref, ac_ref):
        x = s_ref[...][:, 0, :]
        tr = jnp.transpose(x)
        s12 = jnp.sum(tr, axis=0, keepdims=True)
        s128 = jnp.pad(s12, ((0, 0), (0, 128 - nheads)))
        lane = jax.lax.broadcasted_iota(jnp.int32, s128.shape, 1)
        activ = jnp.float32(0.03) * s128
        ac_ref[...] = jnp.exp(jnp.float32(-5.0) * activ)
        ac = ac_ref[...]
        tot = jnp.sum(jnp.where(lane < nheads, ac, jnp.float32(0.0)))
        base = jnp.log(ac)
        last = jnp.log((tot * jnp.float32(99.0) * jnp.float32(batch))
                       * jnp.ones_like(base))
        scores = jnp.where(lane == nheads, last, base)
        scores = jnp.where(lane <= nheads, scores, -jnp.inf)
        scores = scores + gum_ref[...]
        m = jnp.max(scores)
        r = jnp.min(jnp.where(scores == m, lane, jnp.int32(2 ** 30)))
        r_ref[...] = jnp.full(r_ref.shape, r, jnp.int32)
    return body


def _p3_update(block_rows, ntok, nheads):
    def body(r_ref, x_ref, o_ref):
        i = pl.program_id(0)
        r = r_ref[0]
        rr = jnp.minimum(r, nheads - 1)
        t1 = jax.lax.broadcasted_iota(jnp.int32, (1, block_rows, ntok), 1)
        t2 = jax.lax.broadcasted_iota(jnp.int32, (1, block_rows, ntok), 2)
        flat = ((i * block_rows + t1) * ntok + t2) * nheads + rr
        term = _noise_term(flat.astype(jnp.uint32))
        scale = jnp.where(r < nheads, jnp.float32(3.0), jnp.float32(0.0))
        o_ref[...] = x_ref[...] + scale * term
    return body


def kernel(a):
    batch, ntok, ntok2, nheads = a.shape
    av = jnp.transpose(a, (0, 3, 1, 2)).reshape(nheads, ntok, ntok2)

    br1 = 32
    s_hc, acopy = pl.pallas_call(
        _p1_reduce_copy,
        grid=(ntok // br1,),
        in_specs=[pl.BlockSpec((nheads, br1, ntok2), lambda i: (0, i, 0))],
        out_specs=[pl.BlockSpec((nheads, 1, ntok2), lambda i: (0, 0, 0)),
                   pl.BlockSpec((nheads, br1, ntok2), lambda i: (0, i, 0))],
        out_shape=[jax.ShapeDtypeStruct((nheads, 1, ntok2), jnp.float32),
                   jax.ShapeDtypeStruct((nheads, ntok, ntok2), jnp.float32)],
    )(av)

    gum = jnp.zeros((1, 128), jnp.float32)
    gum = gum.at[0, : nheads + 1].set(
        jax.random.gumbel(jax.random.key(42), (nheads + 1,), jnp.float32))

    r_plane = pl.pallas_call(
        _p2_sample(nheads, batch),
        grid=(1,),
        in_specs=[pl.BlockSpec((nheads, 1, ntok2), lambda i: (0, 0, 0)),
                  pl.BlockSpec((1, 128), lambda i: (0, 0))],
        out_specs=pl.BlockSpec((8, 128), lambda i: (0, 0)),
        out_shape=jax.ShapeDtypeStruct((8, 128), jnp.int32),
        scratch_shapes=[pltpu.VMEM((1, 128), jnp.float32)],
    )(s_hc, gum)
    r_arr = r_plane[0, :1]

    br3 = 256
    hmax = nheads - 1

    def _rblock(i, r):
        return (jnp.minimum(r[0], hmax), i, 0)

    outp = pl.pallas_call(
        _p3_update(br3, ntok2, nheads),
        grid_spec=pltpu.PrefetchScalarGridSpec(
            num_scalar_prefetch=1,
            grid=(ntok // br3,),
            in_specs=[pl.BlockSpec((1, br3, ntok2), _rblock)],
            out_specs=pl.BlockSpec((1, br3, ntok2), _rblock),
        ),
        out_shape=jax.ShapeDtypeStruct((nheads, ntok, ntok2), jnp.float32),
        input_output_aliases={1: 0},
    )(r_arr, acopy)

    out = jnp.transpose(outp.reshape(batch, nheads, ntok, ntok2),
                        (0, 2, 3, 1))
    return out


# speedup vs baseline: 30.0470x; 1.2022x over previous
import numpy as np
import jax
import jax.numpy as jnp
from jax.experimental import pallas as pl
from jax.experimental.pallas import tpu as pltpu

_NK1 = 0x03D7B32D
_NK2 = 0xADD083F4

_ROT = ((13, 15, 26, 6), (17, 29, 16, 24))


def _threefry_bits(idx):
    k1 = jnp.uint32(_NK1)
    k2 = jnp.uint32(_NK2)
    ks2 = jnp.uint32(_NK1 ^ _NK2 ^ 0x1BD11BDA)
    ks = (k1, k2, ks2)
    x0 = jnp.full(idx.shape, k1, dtype=jnp.uint32)
    x1 = idx + k2
    for i in range(5):
        for r in _ROT[i % 2]:
            x0 = x0 + x1
            x1 = (x1 << jnp.uint32(r)) | (x1 >> jnp.uint32(32 - r))
            x1 = x1 ^ x0
        x0 = x0 + ks[(i + 1) % 3]
        x1 = x1 + ks[(i + 2) % 3] + jnp.uint32(i + 1)
    return x0 ^ x1


def _noise_term(idx):
    bits = _threefry_bits(idx)
    fb = (bits >> jnp.uint32(9)) | jnp.uint32(0x3F800000)
    f = jax.lax.bitcast_convert_type(fb, jnp.float32) - jnp.float32(1.0)
    lo = jnp.float32(np.nextafter(np.float32(-1.0), np.float32(0.0)))
    hi = jnp.float32(1.0)
    u = jnp.maximum(lo, f * (hi - lo) + lo)
    w = -jnp.log(jnp.float32(1.0) - u * u)
    t = jnp.sqrt(w) - jnp.float32(3.0)
    p = jnp.float32(-0.000200214257)
    for c in (0.000100950558, 0.00134934322, -0.00367342844, 0.00573950773,
              -0.0076224613, 0.00943887047, 1.00167406, 2.83297682):
        p = jnp.float32(c) + p * t
    noise = jnp.float32(np.sqrt(2.0)) * (p * u)
    term = jnp.maximum(noise - jnp.float32(3.0), jnp.float32(0.0))
    return jnp.where(w >= jnp.float32(5.0), term, jnp.float32(0.0))


def _p1_reduce_copy(block_rows, ntok, nheads):
    def body(a_ref, s_ref, o_ref, t0_ref):
        i = pl.program_id(0)
        x = a_ref[...]
        o_ref[...] = x
        part = jnp.sum(jnp.abs(x), axis=1, keepdims=True)

        @pl.when(i == 0)
        def _():
            s_ref[...] = part

        @pl.when(i != 0)
        def _():
            s_ref[...] = s_ref[...] + part

        t1 = jax.lax.broadcasted_iota(jnp.int32, (block_rows, ntok), 0)
        t2 = jax.lax.broadcasted_iota(jnp.int32, (block_rows, ntok), 1)
        flat = ((i * block_rows + t1) * ntok + t2) * nheads
        t0_ref[...] = _noise_term(flat.astype(jnp.uint32))
    return body


def _p2_sample(nheads, batch):
    def body(s_ref, gum_ref, r_ref, ac_ref):
        x = s_ref[...][:, 0, :]
        tr = jnp.transpose(x)
        s12 = jnp.sum(tr, axis=0, keepdims=True)
        s128 = jnp.pad(s12, ((0, 0), (0, 128 - nheads)))
        lane = jax.lax.broadcasted_iota(jnp.int32, s128.shape, 1)
        activ = jnp.float32(0.03) * s128
        ac_ref[...] = jnp.exp(jnp.float32(-5.0) * activ)
        ac = ac_ref[...]
        tot = jnp.sum(jnp.where(lane < nheads, ac, jnp.float32(0.0)))
        base = jnp.log(ac)
        last = jnp.log((tot * jnp.float32(99.0) * jnp.float32(batch))
                       * jnp.ones_like(base))
        scores = jnp.where(lane == nheads, last, base)
        scores = jnp.where(lane <= nheads, scores, -jnp.inf)
        scores = scores + gum_ref[...]
        m = jnp.max(scores)
        r = jnp.min(jnp.where(scores == m, lane, jnp.int32(2 ** 30)))
        r_ref[...] = jnp.full(r_ref.shape, r, jnp.int32)
    return body


def _p3_update(block_rows, ntok, nheads):
    def body(r_ref, x_ref, t0_ref, o_ref):
        i = pl.program_id(0)
        r = r_ref[0]

        @pl.when(r == 0)
        def _():
            o_ref[...] = x_ref[...] + jnp.float32(3.0) * t0_ref[...][None]

        @pl.when(r != 0)
        def _():
            rr = jnp.minimum(r, nheads - 1)
            t1 = jax.lax.broadcasted_iota(jnp.int32, (1, block_rows, ntok), 1)
            t2 = jax.lax.broadcasted_iota(jnp.int32, (1, block_rows, ntok), 2)
            flat = ((i * block_rows + t1) * ntok + t2) * nheads + rr
            term = _noise_term(flat.astype(jnp.uint32))
            scale = jnp.where(r < nheads, jnp.float32(3.0), jnp.float32(0.0))
            o_ref[...] = x_ref[...] + scale * term
    return body


def kernel(a):
    batch, ntok, ntok2, nheads = a.shape
    av = jnp.transpose(a, (0, 3, 1, 2)).reshape(nheads, ntok, ntok2)

    br1 = 32
    s_hc, acopy, term0 = pl.pallas_call(
        _p1_reduce_copy(br1, ntok2, nheads),
        grid=(ntok // br1,),
        in_specs=[pl.BlockSpec((nheads, br1, ntok2), lambda i: (0, i, 0))],
        out_specs=[pl.BlockSpec((nheads, 1, ntok2), lambda i: (0, 0, 0)),
                   pl.BlockSpec((nheads, br1, ntok2), lambda i: (0, i, 0)),
                   pl.BlockSpec((br1, ntok2), lambda i: (i, 0))],
        out_shape=[jax.ShapeDtypeStruct((nheads, 1, ntok2), jnp.float32),
                   jax.ShapeDtypeStruct((nheads, ntok, ntok2), jnp.float32),
                   jax.ShapeDtypeStruct((ntok, ntok2), jnp.float32)],
    )(av)

    gum = jnp.zeros((1, 128), jnp.float32)
    gum = gum.at[0, : nheads + 1].set(
        jax.random.gumbel(jax.random.key(42), (nheads + 1,), jnp.float32))

    r_plane = pl.pallas_call(
        _p2_sample(nheads, batch),
        grid=(1,),
        in_specs=[pl.BlockSpec((nheads, 1, ntok2), lambda i: (0, 0, 0)),
                  pl.BlockSpec((1, 128), lambda i: (0, 0))],
        out_specs=pl.BlockSpec((8, 128), lambda i: (0, 0)),
        out_shape=jax.ShapeDtypeStruct((8, 128), jnp.int32),
        scratch_shapes=[pltpu.VMEM((1, 128), jnp.float32)],
    )(s_hc, gum)
    r_arr = r_plane[0, :1]

    br3 = 256
    hmax = nheads - 1

    def _rblock(i, r):
        return (jnp.minimum(r[0], hmax), i, 0)

    outp = pl.pallas_call(
        _p3_update(br3, ntok2, nheads),
        grid_spec=pltpu.PrefetchScalarGridSpec(
            num_scalar_prefetch=1,
            grid=(ntok // br3,),
            in_specs=[pl.BlockSpec((1, br3, ntok2), _rblock),
                      pl.BlockSpec((br3, ntok2), lambda i, r: (i, 0))],
            out_specs=pl.BlockSpec((1, br3, ntok2), _rblock),
        ),
        out_shape=jax.ShapeDtypeStruct((nheads, ntok, ntok2), jnp.float32),
        input_output_aliases={1: 0},
    )(r_arr, acopy, term0)

    out = jnp.transpose(outp.reshape(batch, nheads, ntok, ntok2),
                        (0, 2, 3, 1))
    return out


# speedup vs baseline: 32.0062x; 1.0652x over previous
import numpy as np
import jax
import jax.numpy as jnp
from jax.experimental import pallas as pl
from jax.experimental.pallas import tpu as pltpu

_NK1 = 0x03D7B32D
_NK2 = 0xADD083F4

_ROT = ((13, 15, 26, 6), (17, 29, 16, 24))


def _threefry_bits(idx):
    k1 = jnp.uint32(_NK1)
    k2 = jnp.uint32(_NK2)
    ks2 = jnp.uint32(_NK1 ^ _NK2 ^ 0x1BD11BDA)
    ks = (k1, k2, ks2)
    x0 = jnp.full(idx.shape, k1, dtype=jnp.uint32)
    x1 = idx + k2
    for i in range(5):
        for r in _ROT[i % 2]:
            x0 = x0 + x1
            x1 = (x1 << jnp.uint32(r)) | (x1 >> jnp.uint32(32 - r))
            x1 = x1 ^ x0
        x0 = x0 + ks[(i + 1) % 3]
        x1 = x1 + ks[(i + 2) % 3] + jnp.uint32(i + 1)
    return x0 ^ x1


def _noise_term(idx):
    bits = _threefry_bits(idx)
    fb = (bits >> jnp.uint32(9)) | jnp.uint32(0x3F800000)
    f = jax.lax.bitcast_convert_type(fb, jnp.float32) - jnp.float32(1.0)
    lo = jnp.float32(np.nextafter(np.float32(-1.0), np.float32(0.0)))
    hi = jnp.float32(1.0)
    u = jnp.maximum(lo, f * (hi - lo) + lo)
    w = -jnp.log(jnp.float32(1.0) - u * u)
    t = jnp.sqrt(w) - jnp.float32(3.0)
    p = jnp.float32(-0.000200214257)
    for c in (0.000100950558, 0.00134934322, -0.00367342844, 0.00573950773,
              -0.0076224613, 0.00943887047, 1.00167406, 2.83297682):
        p = jnp.float32(c) + p * t
    noise = jnp.float32(np.sqrt(2.0)) * (p * u)
    term = jnp.maximum(noise - jnp.float32(3.0), jnp.float32(0.0))
    return jnp.where(w >= jnp.float32(5.0), term, jnp.float32(0.0))


def _p1_reduce_copy(block_rows, ntok, nheads):
    def body(a_ref, s_ref, o_ref):
        i = pl.program_id(0)
        x = a_ref[...]
        o_ref[...] = x
        part = jnp.sum(jnp.abs(x), axis=1, keepdims=True)

        @pl.when(i == 0)
        def _():
            s_ref[...] = part

        @pl.when(i != 0)
        def _():
            s_ref[...] = s_ref[...] + part

        t1 = jax.lax.broadcasted_iota(jnp.int32, (block_rows, ntok), 0)
        t2 = jax.lax.broadcasted_iota(jnp.int32, (block_rows, ntok), 1)
        flat = ((i * block_rows + t1) * ntok + t2) * nheads
        term = _noise_term(flat.astype(jnp.uint32))
        o_ref[0, :, :] = x[0] + jnp.float32(3.0) * term
    return body


def _p2_sample(nheads, batch):
    def body(s_ref, gum_ref, r_ref, ac_ref):
        x = s_ref[...][:, 0, :]
        tr = jnp.transpose(x)
        s12 = jnp.sum(tr, axis=0, keepdims=True)
        s128 = jnp.pad(s12, ((0, 0), (0, 128 - nheads)))
        lane = jax.lax.broadcasted_iota(jnp.int32, s128.shape, 1)
        activ = jnp.float32(0.03) * s128
        ac_ref[...] = jnp.exp(jnp.float32(-5.0) * activ)
        ac = ac_ref[...]
        tot = jnp.sum(jnp.where(lane < nheads, ac, jnp.float32(0.0)))
        base = jnp.log(ac)
        last = jnp.log((tot * jnp.float32(99.0) * jnp.float32(batch))
                       * jnp.ones_like(base))
        scores = jnp.where(lane == nheads, last, base)
        scores = jnp.where(lane <= nheads, scores, -jnp.inf)
        scores = scores + gum_ref[...]
        m = jnp.max(scores)
        r = jnp.min(jnp.where(scores == m, lane, jnp.int32(2 ** 30)))
        r_ref[...] = jnp.full(r_ref.shape, r, jnp.int32)
    return body


def _p3_update(block_rows, ntok, nheads):
    def body(r_ref, src_ref, x_ref, o_ref):
        j = pl.program_id(0)
        i = pl.program_id(1)
        r = r_ref[0]

        @pl.when(r == 0)
        def _():
            o_ref[...] = x_ref[...]

        @pl.when((r != 0) & (j == 0))
        def _():
            o_ref[...] = src_ref[...]

        @pl.when((r != 0) & (j == 1))
        def _():
            rr = jnp.minimum(r, nheads - 1)
            t1 = jax.lax.broadcasted_iota(jnp.int32, (1, block_rows, ntok), 1)
            t2 = jax.lax.broadcasted_iota(jnp.int32, (1, block_rows, ntok), 2)
            flat = ((i * block_rows + t1) * ntok + t2) * nheads + rr
            term = _noise_term(flat.astype(jnp.uint32))
            scale = jnp.where(r < nheads, jnp.float32(3.0), jnp.float32(0.0))
            o_ref[...] = src_ref[...] + scale * term
    return body


def kernel(a):
    batch, ntok, ntok2, nheads = a.shape
    av = jnp.transpose(a, (0, 3, 1, 2)).reshape(nheads, ntok, ntok2)

    br1 = 32
    s_hc, acopy = pl.pallas_call(
        _p1_reduce_copy(br1, ntok2, nheads),
        grid=(ntok // br1,),
        in_specs=[pl.BlockSpec((nheads, br1, ntok2), lambda i: (0, i, 0))],
        out_specs=[pl.BlockSpec((nheads, 1, ntok2), lambda i: (0, 0, 0)),
                   pl.BlockSpec((nheads, br1, ntok2), lambda i: (0, i, 0))],
        out_shape=[jax.ShapeDtypeStruct((nheads, 1, ntok2), jnp.float32),
                   jax.ShapeDtypeStruct((nheads, ntok, ntok2), jnp.float32)],
    )(av)

    gum = jnp.zeros((1, 128), jnp.float32)
    gum = gum.at[0, : nheads + 1].set(
        jax.random.gumbel(jax.random.key(42), (nheads + 1,), jnp.float32))

    r_plane = pl.pallas_call(
        _p2_sample(nheads, batch),
        grid=(1,),
        in_specs=[pl.BlockSpec((nheads, 1, ntok2), lambda i: (0, 0, 0)),
                  pl.BlockSpec((1, 128), lambda i: (0, 0))],
        out_specs=pl.BlockSpec((8, 128), lambda i: (0, 0)),
        out_shape=jax.ShapeDtypeStruct((8, 128), jnp.int32),
        scratch_shapes=[pltpu.VMEM((1, 128), jnp.float32)],
    )(s_hc, gum)
    r_arr = r_plane[0, :1]

    br3 = 256
    hmax = nheads - 1

    def _rblock(j, i, r):
        plane = jnp.where(r[0] == 0, 0,
                          jnp.where(j == 0, 0, jnp.minimum(r[0], hmax)))
        blk = jnp.where(r[0] == 0, 0, i)
        return (plane, blk, 0)

    outp = pl.pallas_call(
        _p3_update(br3, ntok2, nheads),
        grid_spec=pltpu.PrefetchScalarGridSpec(
            num_scalar_prefetch=1,
            grid=(2, ntok // br3),
            in_specs=[pl.BlockSpec((1, br3, ntok2), _rblock),
                      pl.BlockSpec((1, br3, ntok2), _rblock)],
            out_specs=pl.BlockSpec((1, br3, ntok2), _rblock),
        ),
        out_shape=jax.ShapeDtypeStruct((nheads, ntok, ntok2), jnp.float32),
        input_output_aliases={2: 0},
    )(r_arr, av, acopy)

    out = jnp.transpose(outp.reshape(batch, nheads, ntok, ntok2),
                        (0, 2, 3, 1))
    return out


# speedup vs baseline: 35.9670x; 1.1238x over previous
import numpy as np
import jax
import jax.numpy as jnp
from jax.experimental import pallas as pl
from jax.experimental.pallas import tpu as pltpu

_NK1 = 0x03D7B32D
_NK2 = 0xADD083F4

_ROT = ((13, 15, 26, 6), (17, 29, 16, 24))


def _threefry_bits(idx):
    k1 = jnp.uint32(_NK1)
    k2 = jnp.uint32(_NK2)
    ks2 = jnp.uint32(_NK1 ^ _NK2 ^ 0x1BD11BDA)
    ks = (k1, k2, ks2)
    x0 = jnp.full(idx.shape, k1, dtype=jnp.uint32)
    x1 = idx + k2
    for i in range(5):
        for r in _ROT[i % 2]:
            x0 = x0 + x1
            x1 = (x1 << jnp.uint32(r)) | (x1 >> jnp.uint32(32 - r))
            x1 = x1 ^ x0
        x0 = x0 + ks[(i + 1) % 3]
        x1 = x1 + ks[(i + 2) % 3] + jnp.uint32(i + 1)
    return x0 ^ x1


def _noise_term(idx):
    bits = _threefry_bits(idx)
    fb = (bits >> jnp.uint32(9)) | jnp.uint32(0x3F800000)
    f = jax.lax.bitcast_convert_type(fb, jnp.float32) - jnp.float32(1.0)
    lo = jnp.float32(np.nextafter(np.float32(-1.0), np.float32(0.0)))
    hi = jnp.float32(1.0)
    u = jnp.maximum(lo, f * (hi - lo) + lo)
    w = -jnp.log(jnp.float32(1.0) - u * u)
    t = jnp.sqrt(w) - jnp.float32(3.0)
    p = jnp.float32(-0.000200214257)
    for c in (0.000100950558, 0.00134934322, -0.00367342844, 0.00573950773,
              -0.0076224613, 0.00943887047, 1.00167406, 2.83297682):
        p = jnp.float32(c) + p * t
    noise = jnp.float32(np.sqrt(2.0)) * (p * u)
    term = jnp.maximum(noise - jnp.float32(3.0), jnp.float32(0.0))
    return jnp.where(w >= jnp.float32(5.0), term, jnp.float32(0.0))


def _p1_reduce_copy(block_rows, ntok, nheads):
    def body(a_ref, s_ref, o_ref):
        i = pl.program_id(0)
        x = a_ref[...]
        o_ref[...] = x
        part = jnp.sum(jnp.abs(x), axis=1, keepdims=True)

        @pl.when(i == 0)
        def _():
            s_ref[...] = part

        @pl.when(i != 0)
        def _():
            s_ref[...] = s_ref[...] + part

        t1 = jax.lax.broadcasted_iota(jnp.int32, (block_rows, ntok), 0)
        t2 = jax.lax.broadcasted_iota(jnp.int32, (block_rows, ntok), 1)
        flat = ((i * block_rows + t1) * ntok + t2) * nheads
        term = _noise_term(flat.astype(jnp.uint32))
        o_ref[0, :, :] = x[0] + jnp.float32(3.0) * term
    return body


def _p2_sample(nheads, batch):
    def body(s_ref, gum_ref, r_ref, ac_ref):
        x = s_ref[...][:, 0, :]
        tr = jnp.transpose(x)
        s12 = jnp.sum(tr, axis=0, keepdims=True)
        s128 = jnp.pad(s12, ((0, 0), (0, 128 - nheads)))
        lane = jax.lax.broadcasted_iota(jnp.int32, s128.shape, 1)
        activ = jnp.float32(0.03) * s128
        ac_ref[...] = jnp.exp(jnp.float32(-5.0) * activ)
        ac = ac_ref[...]
        tot = jnp.sum(jnp.where(lane < nheads, ac, jnp.float32(0.0)))
        base = jnp.log(ac)
        last = jnp.log((tot * jnp.float32(99.0) * jnp.float32(batch))
                       * jnp.ones_like(base))
        scores = jnp.where(lane == nheads, last, base)
        scores = jnp.where(lane <= nheads, scores, -jnp.inf)
        scores = scores + gum_ref[...]
        m = jnp.max(scores)
        r = jnp.min(jnp.where(scores == m, lane, jnp.int32(2 ** 30)))
        r_ref[...] = jnp.full(r_ref.shape, r, jnp.int32)
    return body


def _p3_update(block_rows, ntok, nheads):
    def body(r_ref, src_ref, x_ref, o_ref):
        j = pl.program_id(0)
        i = pl.program_id(1)
        r = r_ref[0]

        @pl.when(r == 0)
        def _():
            o_ref[...] = x_ref[...]

        @pl.when((r != 0) & (j == 0))
        def _():
            o_ref[...] = src_ref[...]

        @pl.when((r != 0) & (j == 1))
        def _():
            rr = jnp.minimum(r, nheads - 1)
            t1 = jax.lax.broadcasted_iota(jnp.int32, (1, block_rows, ntok), 1)
            t2 = jax.lax.broadcasted_iota(jnp.int32, (1, block_rows, ntok), 2)
            flat = ((i * block_rows + t1) * ntok + t2) * nheads + rr
            term = _noise_term(flat.astype(jnp.uint32))
            scale = jnp.where(r < nheads, jnp.float32(3.0), jnp.float32(0.0))
            o_ref[...] = src_ref[...] + scale * term
    return body


def kernel(a):
    batch, ntok, ntok2, nheads = a.shape
    av = jnp.transpose(a, (0, 3, 1, 2)).reshape(nheads, ntok, ntok2)

    br1 = 64
    s_hc, acopy = pl.pallas_call(
        _p1_reduce_copy(br1, ntok2, nheads),
        grid=(ntok // br1,),
        in_specs=[pl.BlockSpec((nheads, br1, ntok2), lambda i: (0, i, 0))],
        out_specs=[pl.BlockSpec((nheads, 1, ntok2), lambda i: (0, 0, 0)),
                   pl.BlockSpec((nheads, br1, ntok2), lambda i: (0, i, 0))],
        out_shape=[jax.ShapeDtypeStruct((nheads, 1, ntok2), jnp.float32),
                   jax.ShapeDtypeStruct((nheads, ntok, ntok2), jnp.float32)],
    )(av)

    gum = jnp.zeros((1, 128), jnp.float32)
    gum = gum.at[0, : nheads + 1].set(
        jax.random.gumbel(jax.random.key(42), (nheads + 1,), jnp.float32))

    r_plane = pl.pallas_call(
        _p2_sample(nheads, batch),
        grid=(1,),
        in_specs=[pl.BlockSpec((nheads, 1, ntok2), lambda i: (0, 0, 0)),
                  pl.BlockSpec((1, 128), lambda i: (0, 0))],
        out_specs=pl.BlockSpec((8, 128), lambda i: (0, 0)),
        out_shape=jax.ShapeDtypeStruct((8, 128), jnp.int32),
        scratch_shapes=[pltpu.VMEM((1, 128), jnp.float32)],
    )(s_hc, gum)
    r_arr = r_plane[0, :1]

    br3 = 256
    hmax = nheads - 1

    def _rblock(j, i, r):
        plane = jnp.where(r[0] == 0, 0,
                          jnp.where(j == 0, 0, jnp.minimum(r[0], hmax)))
        blk = jnp.where(r[0] == 0, 0, i)
        return (plane, blk, 0)

    outp = pl.pallas_call(
        _p3_update(br3, ntok2, nheads),
        grid_spec=pltpu.PrefetchScalarGridSpec(
            num_scalar_prefetch=1,
            grid=(2, ntok // br3),
            in_specs=[pl.BlockSpec((1, br3, ntok2), _rblock),
                      pl.BlockSpec((1, br3, ntok2), _rblock)],
            out_specs=pl.BlockSpec((1, br3, ntok2), _rblock),
        ),
        out_shape=jax.ShapeDtypeStruct((nheads, ntok, ntok2), jnp.float32),
        input_output_aliases={2: 0},
    )(r_arr, av, acopy)

    out = jnp.transpose(outp.reshape(batch, nheads, ntok, ntok2),
                        (0, 2, 3, 1))
    return out
